# baseline (speedup 1.0000x reference)
import functools

import jax
import jax.numpy as jnp
from jax import lax
from jax.experimental import pallas as pl
from jax.experimental.pallas import tpu as pltpu
from jax.experimental.pallas import tpu_sc as plsc

N = 10000
E = 320000
D = 128
ALPHA = 0.2

NP = 10240
NC = 2
NS = 16
NW = NC * NS
EPW = E // NW
CPW = D // NW
LANES = 16

BN = 1024
NBLK = NP // BN

CHUNK = 4000
NCHUNK = E // CHUNK


def _prep_body(x_ref, w_ref, a_ref, s1_ref, s2_ref, wxp_ref):
    xb = x_ref[...]
    wm = w_ref[...]
    wxt = lax.dot_general(wm, xb, (((0,), (1,)), ((), ())),
                          preferred_element_type=jnp.float32)
    a1 = a_ref[:D, :]
    a2 = a_ref[D:, :]
    s1 = lax.dot_general(a1, wxt, (((0,), (0,)), ((), ())),
                         preferred_element_type=jnp.float32)
    s2 = lax.dot_general(a2, wxt, (((0,), (0,)), ((), ())),
                         preferred_element_type=jnp.float32)
    s1_ref[...] = s1[0]
    s2_ref[...] = s2[0]
    lo = lax.bitcast_convert_type(
        wxt[:D // 2, :].astype(jnp.bfloat16), jnp.uint16).astype(jnp.uint32)
    hi = lax.bitcast_convert_type(
        wxt[D // 2:, :].astype(jnp.bfloat16), jnp.uint16).astype(jnp.uint32)
    wxp_ref[...] = lax.bitcast_convert_type(lo | (hi << 16), jnp.int32)


def _prep(xp, W, a):
    return pl.pallas_call(
        _prep_body,
        grid=(NBLK,),
        in_specs=[
            pl.BlockSpec((BN, D), lambda i: (i, 0)),
            pl.BlockSpec((D, D), lambda i: (0, 0)),
            pl.BlockSpec((2 * D, 1), lambda i: (0, 0)),
        ],
        out_specs=[
            pl.BlockSpec((BN,), lambda i: (i,)),
            pl.BlockSpec((BN,), lambda i: (i,)),
            pl.BlockSpec((D // 2, BN), lambda i: (0, i)),
        ],
        out_shape=[
            jax.ShapeDtypeStruct((NP,), jnp.float32),
            jax.ShapeDtypeStruct((NP,), jnp.float32),
            jax.ShapeDtypeStruct((D // 2, NP), jnp.int32),
        ],
    )(xp, W, a)


def _phase_a_body(s1_hbm, s2_hbm, src_hbm, dst_hbm,
                  expv_hbm, sums_hbm, sdp_hbm,
                  s1_v, s2_v, src_v, dst_v, exp_v, sum_v, sd_v):
    wid = lax.axis_index("c") * NS + lax.axis_index("s")
    base = wid * EPW

    pltpu.sync_copy(s1_hbm, s1_v)
    pltpu.sync_copy(s2_hbm, s2_v)
    pltpu.sync_copy(src_hbm.at[pl.ds(base, EPW)], src_v)
    pltpu.sync_copy(dst_hbm.at[pl.ds(base, EPW)], dst_v)

    zeros = jnp.zeros((LANES,), jnp.float32)

    @pl.loop(0, NP // LANES, unroll=8)
    def _zero(j):
        sum_v[pl.ds(j * LANES, LANES)] = zeros

    @plsc.parallel_loop(0, EPW // LANES, unroll=8)
    def _edges(i):
        off = i * LANES
        s16 = src_v[pl.ds(off, LANES)]
        d16 = dst_v[pl.ds(off, LANES)]
        sd_v[pl.ds(off, LANES)] = s16 | (d16 << 16)
        v1 = plsc.load_gather(s1_v, [s16])
        v2 = plsc.load_gather(s2_v, [d16])
        t = v1 + v2
        e = jnp.maximum(t, t * ALPHA)
        ev = jnp.exp(e)
        exp_v[pl.ds(off, LANES)] = ev
        plsc.addupdate_scatter(sum_v, [s16], ev)

    pltpu.sync_copy(exp_v, expv_hbm.at[pl.ds(base, EPW)])
    pltpu.sync_copy(sum_v, sums_hbm.at[wid])
    pltpu.sync_copy(sd_v, sdp_hbm.at[pl.ds(base, EPW)])


def _phase_a(s1, s2, src, dst):
    mesh = plsc.VectorSubcoreMesh(core_axis_name="c", subcore_axis_name="s")
    return pl.kernel(
        _phase_a_body,
        out_type=[
            jax.ShapeDtypeStruct((E,), jnp.float32),
            jax.ShapeDtypeStruct((NW, NP), jnp.float32),
            jax.ShapeDtypeStruct((E,), jnp.int32),
        ],
        mesh=mesh,
        compiler_params=pltpu.CompilerParams(needs_layout_passes=False),
        scratch_types=[
            pltpu.VMEM((NP,), jnp.float32),
            pltpu.VMEM((NP,), jnp.float32),
            pltpu.VMEM((EPW,), jnp.int32),
            pltpu.VMEM((EPW,), jnp.int32),
            pltpu.VMEM((EPW,), jnp.float32),
            pltpu.VMEM((NP,), jnp.float32),
            pltpu.VMEM((EPW,), jnp.int32),
        ],
    )(s1, s2, src, dst)


def _phase_b_body(wxp_hbm, sdp_hbm, expv_hbm,
                  ht_hbm,
                  wx_v, h_v, h2_v, sd0, sd1, ex0, ex1, sem_a, sem_b):
    wid = lax.axis_index("c") * NS + lax.axis_index("s")
    p0 = wid * (CPW // 2)

    for p in range(CPW // 2):
        pltpu.sync_copy(wxp_hbm.at[p0 + p], wx_v.at[pl.ds(p * NP, NP)])

    zeros = jnp.zeros((LANES,), jnp.float32)

    @pl.loop(0, CPW * NP // LANES, unroll=8)
    def _zero(j):
        h_v[pl.ds(j * LANES, LANES)] = zeros
        h2_v[pl.ds(j * LANES, LANES)] = zeros

    def process(sd_v, exp_v):
        @plsc.parallel_loop(0, CHUNK // (2 * LANES), unroll=4)
        def _edges(i):
            for half, acc in ((0, h_v), (1, h2_v)):
                off = (2 * i + half) * LANES
                sd = sd_v[pl.ds(off, LANES)]
                e16 = exp_v[pl.ds(off, LANES)]
                s16 = sd & 0xFFFF
                d16 = lax.shift_right_logical(sd, 16)
                gs = [plsc.load_gather(wx_v, [d16 + (p * NP)])
                      for p in range(CPW // 2)]
                vals = []
                for g in gs:
                    lo, hi = plsc.unpack(plsc.bitcast(g, jnp.bfloat16),
                                         format=plsc.PackFormat.INTERLEAVED)
                    vals += [e16 * lo, e16 * hi]
                for c in range(CPW):
                    plsc.addupdate_scatter(acc, [s16 + (c * NP)], vals[c])

    pltpu.async_copy(sdp_hbm.at[pl.ds(0, CHUNK)], sd0, sem_a)
    pltpu.async_copy(expv_hbm.at[pl.ds(0, CHUNK)], ex0, sem_a)

    @pl.loop(0, NCHUNK // 2)
    def _pair(m):
        k0 = 2 * m
        d1 = pltpu.async_copy(
            sdp_hbm.at[pl.ds((k0 + 1) * CHUNK, CHUNK)], sd1, sem_b)
        d2 = pltpu.async_copy(
            expv_hbm.at[pl.ds((k0 + 1) * CHUNK, CHUNK)], ex1, sem_b)
        pltpu.make_async_copy(
            sdp_hbm.at[pl.ds(k0 * CHUNK, CHUNK)], sd0, sem_a).wait()
        pltpu.make_async_copy(
            expv_hbm.at[pl.ds(k0 * CHUNK, CHUNK)], ex0, sem_a).wait()
        process(sd0, ex0)

        @pl.when(k0 + 2 < NCHUNK)
        def _prefetch():
            pltpu.async_copy(
                sdp_hbm.at[pl.ds((k0 + 2) * CHUNK, CHUNK)], sd0, sem_a)
            pltpu.async_copy(
                expv_hbm.at[pl.ds((k0 + 2) * CHUNK, CHUNK)], ex0, sem_a)

        d1.wait()
        d2.wait()
        process(sd1, ex1)

    @pl.loop(0, CPW * NP // LANES, unroll=8)
    def _merge(j):
        sl = pl.ds(j * LANES, LANES)
        h_v[sl] = h_v[sl] + h2_v[sl]

    for p in range(CPW // 2):
        pltpu.sync_copy(h_v.at[pl.ds((2 * p) * NP, NP)],
                        ht_hbm.at[p0 + p])
        pltpu.sync_copy(h_v.at[pl.ds((2 * p + 1) * NP, NP)],
                        ht_hbm.at[p0 + p + D // 2])


def _phase_b(wxp, sdp, expv):
    mesh = plsc.VectorSubcoreMesh(core_axis_name="c", subcore_axis_name="s")
    return pl.kernel(
        _phase_b_body,
        out_type=jax.ShapeDtypeStruct((D, NP), jnp.float32),
        mesh=mesh,
        compiler_params=pltpu.CompilerParams(needs_layout_passes=False),
        scratch_types=[
            pltpu.VMEM(((CPW // 2) * NP,), jnp.int32),
            pltpu.VMEM((CPW * NP,), jnp.float32),
            pltpu.VMEM((CPW * NP,), jnp.float32),
            pltpu.VMEM((CHUNK,), jnp.int32),
            pltpu.VMEM((CHUNK,), jnp.int32),
            pltpu.VMEM((CHUNK,), jnp.float32),
            pltpu.VMEM((CHUNK,), jnp.float32),
            pltpu.SemaphoreType.DMA,
            pltpu.SemaphoreType.DMA,
        ],
    )(wxp, sdp, expv)


def _finish_body(ht_ref, sums_ref, out_ref):
    s = jnp.sum(sums_ref[...], axis=0)
    s = jnp.where(s == 0.0, 1.0, s)
    att = ht_ref[...] / s[None, :]
    z = att.T
    out_ref[...] = jnp.where(z > 0.0, z, jnp.exp(z) - 1.0)


def _finish(ht, sums):
    return pl.pallas_call(
        _finish_body,
        grid=(NBLK,),
        in_specs=[
            pl.BlockSpec((D, BN), lambda i: (0, i)),
            pl.BlockSpec((NW, BN), lambda i: (0, i)),
        ],
        out_specs=pl.BlockSpec((BN, D), lambda i: (i, 0)),
        out_shape=jax.ShapeDtypeStruct((N, D), jnp.float32),
    )(ht, sums)


def kernel(x, edge_index, W, a):
    src = edge_index[0]
    dst = edge_index[1]
    xp = jnp.zeros((NP, D), jnp.float32).at[:N].set(x)
    s1, s2, wxp = _prep(xp, W, a)
    expv, sums, sdp = _phase_a(s1, s2, src, dst)
    ht = _phase_b(wxp, sdp, expv)
    return _finish(ht, sums)


# speedup vs baseline: 1.1011x; 1.1011x over previous
import functools

import jax
import jax.numpy as jnp
from jax import lax
from jax.experimental import pallas as pl
from jax.experimental.pallas import tpu as pltpu
from jax.experimental.pallas import tpu_sc as plsc

N = 10000
E = 320000
D = 128
ALPHA = 0.2

NP = 10240
NC = 2
NS = 16
NW = NC * NS
EPW = E // NW
CPW = D // NW
LANES = 16

BN = 1024
NBLK = NP // BN

CHUNK = 16000
NCHUNK = E // CHUNK


def _prep_body(x_ref, w_ref, a_ref, s1_ref, s2_ref, wxp_ref):
    xb = x_ref[...]
    wm = w_ref[...]
    wxt = lax.dot_general(wm, xb, (((0,), (1,)), ((), ())),
                          preferred_element_type=jnp.float32)
    a1 = a_ref[:D, :]
    a2 = a_ref[D:, :]
    s1 = lax.dot_general(a1, wxt, (((0,), (0,)), ((), ())),
                         preferred_element_type=jnp.float32)
    s2 = lax.dot_general(a2, wxt, (((0,), (0,)), ((), ())),
                         preferred_element_type=jnp.float32)
    s1_ref[...] = s1[0]
    s2_ref[...] = s2[0]
    lo = lax.bitcast_convert_type(
        wxt[:D // 2, :].astype(jnp.bfloat16), jnp.uint16).astype(jnp.uint32)
    hi = lax.bitcast_convert_type(
        wxt[D // 2:, :].astype(jnp.bfloat16), jnp.uint16).astype(jnp.uint32)
    wxp_ref[...] = lax.bitcast_convert_type(lo | (hi << 16), jnp.int32)


def _prep(xp, W, a):
    return pl.pallas_call(
        _prep_body,
        grid=(NBLK,),
        in_specs=[
            pl.BlockSpec((BN, D), lambda i: (i, 0)),
            pl.BlockSpec((D, D), lambda i: (0, 0)),
            pl.BlockSpec((2 * D, 1), lambda i: (0, 0)),
        ],
        out_specs=[
            pl.BlockSpec((BN,), lambda i: (i,)),
            pl.BlockSpec((BN,), lambda i: (i,)),
            pl.BlockSpec((D // 2, BN), lambda i: (0, i)),
        ],
        out_shape=[
            jax.ShapeDtypeStruct((NP,), jnp.float32),
            jax.ShapeDtypeStruct((NP,), jnp.float32),
            jax.ShapeDtypeStruct((D // 2, NP), jnp.int32),
        ],
    )(xp, W, a)


def _phase_a_body(s1_hbm, s2_hbm, src_hbm, dst_hbm,
                  expv_hbm, sums_hbm, sdp_hbm,
                  s1_v, s2_v, src_v, dst_v, exp_v, sum_v, sd_v):
    wid = lax.axis_index("c") * NS + lax.axis_index("s")
    base = wid * EPW

    pltpu.sync_copy(s1_hbm, s1_v)
    pltpu.sync_copy(s2_hbm, s2_v)
    pltpu.sync_copy(src_hbm.at[pl.ds(base, EPW)], src_v)
    pltpu.sync_copy(dst_hbm.at[pl.ds(base, EPW)], dst_v)

    zeros = jnp.zeros((LANES,), jnp.float32)

    @pl.loop(0, NP // LANES, unroll=8)
    def _zero(j):
        sum_v[pl.ds(j * LANES, LANES)] = zeros

    @plsc.parallel_loop(0, EPW // LANES, unroll=8)
    def _edges(i):
        off = i * LANES
        s16 = src_v[pl.ds(off, LANES)]
        d16 = dst_v[pl.ds(off, LANES)]
        sd_v[pl.ds(off, LANES)] = s16 | (d16 << 16)
        v1 = plsc.load_gather(s1_v, [s16])
        v2 = plsc.load_gather(s2_v, [d16])
        t = v1 + v2
        e = jnp.maximum(t, t * ALPHA)
        ev = jnp.exp(e)
        exp_v[pl.ds(off, LANES)] = ev
        plsc.addupdate_scatter(sum_v, [s16], ev)

    pltpu.sync_copy(exp_v, expv_hbm.at[pl.ds(base, EPW)])
    pltpu.sync_copy(sum_v, sums_hbm.at[wid])
    pltpu.sync_copy(sd_v, sdp_hbm.at[pl.ds(base, EPW)])


def _phase_a(s1, s2, src, dst):
    mesh = plsc.VectorSubcoreMesh(core_axis_name="c", subcore_axis_name="s")
    return pl.kernel(
        _phase_a_body,
        out_type=[
            jax.ShapeDtypeStruct((E,), jnp.float32),
            jax.ShapeDtypeStruct((NW, NP), jnp.float32),
            jax.ShapeDtypeStruct((E,), jnp.int32),
        ],
        mesh=mesh,
        compiler_params=pltpu.CompilerParams(needs_layout_passes=False),
        scratch_types=[
            pltpu.VMEM((NP,), jnp.float32),
            pltpu.VMEM((NP,), jnp.float32),
            pltpu.VMEM((EPW,), jnp.int32),
            pltpu.VMEM((EPW,), jnp.int32),
            pltpu.VMEM((EPW,), jnp.float32),
            pltpu.VMEM((NP,), jnp.float32),
            pltpu.VMEM((EPW,), jnp.int32),
        ],
    )(s1, s2, src, dst)


def _phase_b_body(wxp_hbm, sdp_hbm, expv_hbm,
                  ht_hbm,
                  wx_v, h_v, sd0, sd1, ex0, ex1, sem_a, sem_b):
    wid = lax.axis_index("c") * NS + lax.axis_index("s")
    p0 = wid * (CPW // 2)

    for p in range(CPW // 2):
        pltpu.sync_copy(wxp_hbm.at[p0 + p], wx_v.at[pl.ds(p * NP, NP)])

    zeros = jnp.zeros((LANES,), jnp.float32)

    @pl.loop(0, CPW * NP // LANES, unroll=8)
    def _zero(j):
        h_v[pl.ds(j * LANES, LANES)] = zeros

    def process(sd_v, exp_v):
        @plsc.parallel_loop(0, CHUNK // LANES, unroll=8)
        def _edges(i):
            off = i * LANES
            sd = sd_v[pl.ds(off, LANES)]
            e16 = exp_v[pl.ds(off, LANES)]
            s16 = sd & 0xFFFF
            d16 = lax.shift_right_logical(sd, 16)
            gs = [plsc.load_gather(wx_v, [d16 + (p * NP)])
                  for p in range(CPW // 2)]
            vals = []
            for g in gs:
                lo, hi = plsc.unpack(plsc.bitcast(g, jnp.bfloat16),
                                     format=plsc.PackFormat.INTERLEAVED)
                vals += [e16 * lo, e16 * hi]
            for c in range(CPW):
                plsc.addupdate_scatter(h_v, [s16 + (c * NP)], vals[c])

    pltpu.async_copy(sdp_hbm.at[pl.ds(0, CHUNK)], sd0, sem_a)
    pltpu.async_copy(expv_hbm.at[pl.ds(0, CHUNK)], ex0, sem_a)

    @pl.loop(0, NCHUNK // 2)
    def _pair(m):
        k0 = 2 * m
        d1 = pltpu.async_copy(
            sdp_hbm.at[pl.ds((k0 + 1) * CHUNK, CHUNK)], sd1, sem_b)
        d2 = pltpu.async_copy(
            expv_hbm.at[pl.ds((k0 + 1) * CHUNK, CHUNK)], ex1, sem_b)
        pltpu.make_async_copy(
            sdp_hbm.at[pl.ds(k0 * CHUNK, CHUNK)], sd0, sem_a).wait()
        pltpu.make_async_copy(
            expv_hbm.at[pl.ds(k0 * CHUNK, CHUNK)], ex0, sem_a).wait()
        process(sd0, ex0)

        @pl.when(k0 + 2 < NCHUNK)
        def _prefetch():
            pltpu.async_copy(
                sdp_hbm.at[pl.ds((k0 + 2) * CHUNK, CHUNK)], sd0, sem_a)
            pltpu.async_copy(
                expv_hbm.at[pl.ds((k0 + 2) * CHUNK, CHUNK)], ex0, sem_a)

        d1.wait()
        d2.wait()
        process(sd1, ex1)

    for p in range(CPW // 2):
        pltpu.sync_copy(h_v.at[pl.ds((2 * p) * NP, NP)],
                        ht_hbm.at[p0 + p])
        pltpu.sync_copy(h_v.at[pl.ds((2 * p + 1) * NP, NP)],
                        ht_hbm.at[p0 + p + D // 2])


def _phase_b(wxp, sdp, expv):
    mesh = plsc.VectorSubcoreMesh(core_axis_name="c", subcore_axis_name="s")
    return pl.kernel(
        _phase_b_body,
        out_type=jax.ShapeDtypeStruct((D, NP), jnp.float32),
        mesh=mesh,
        compiler_params=pltpu.CompilerParams(needs_layout_passes=False),
        scratch_types=[
            pltpu.VMEM(((CPW // 2) * NP,), jnp.int32),
            pltpu.VMEM((CPW * NP,), jnp.float32),
            pltpu.VMEM((CHUNK,), jnp.int32),
            pltpu.VMEM((CHUNK,), jnp.int32),
            pltpu.VMEM((CHUNK,), jnp.float32),
            pltpu.VMEM((CHUNK,), jnp.float32),
            pltpu.SemaphoreType.DMA,
            pltpu.SemaphoreType.DMA,
        ],
    )(wxp, sdp, expv)


def _finish_body(ht_ref, sums_ref, out_ref):
    s = jnp.sum(sums_ref[...], axis=0)
    s = jnp.where(s == 0.0, 1.0, s)
    att = ht_ref[...] / s[None, :]
    z = att.T
    out_ref[...] = jnp.where(z > 0.0, z, jnp.exp(z) - 1.0)


def _finish(ht, sums):
    return pl.pallas_call(
        _finish_body,
        grid=(NBLK,),
        in_specs=[
            pl.BlockSpec((D, BN), lambda i: (0, i)),
            pl.BlockSpec((NW, BN), lambda i: (0, i)),
        ],
        out_specs=pl.BlockSpec((BN, D), lambda i: (i, 0)),
        out_shape=jax.ShapeDtypeStruct((N, D), jnp.float32),
    )(ht, sums)


def kernel(x, edge_index, W, a):
    src = edge_index[0]
    dst = edge_index[1]
    xp = jnp.zeros((NP, D), jnp.float32).at[:N].set(x)
    s1, s2, wxp = _prep(xp, W, a)
    expv, sums, sdp = _phase_a(s1, s2, src, dst)
    ht = _phase_b(wxp, sdp, expv)
    return _finish(ht, sums)
